# baseline (speedup 1.0000x reference)
import functools

import jax
import jax.numpy as jnp
from jax import lax
from jax.experimental import pallas as pl
from jax.experimental.pallas import tpu as pltpu
from jax.experimental.pallas import tpu_sc as plsc

_E = 64
_TOPK = 8
_BLK = 1024

_NC = 2
_NS = 16
_NW = _NC * _NS
_L = 16


def _scores_block(x_ref, w_ref, scores_ref):
    logits = jax.lax.dot_general(
        x_ref[...], w_ref[...], (((1,), (1,)), ((), ())),
        preferred_element_type=jnp.float32)
    m = jnp.max(logits, axis=-1, keepdims=True)
    e = jnp.exp(logits - m)
    scores_ref[...] = e / jnp.sum(e, axis=-1, keepdims=True)


def _tc_scores(xf, W):
    t, hs = xf.shape
    return pl.pallas_call(
        _scores_block,
        grid=(t // _BLK,),
        in_specs=[
            pl.BlockSpec((_BLK, hs), lambda i: (i, 0)),
            pl.BlockSpec((_E, hs), lambda i: (0, 0)),
        ],
        out_specs=pl.BlockSpec((_BLK, _E), lambda i: (i, 0)),
        out_shape=jax.ShapeDtypeStruct((t, _E), jnp.float32),
    )(xf, W)


def _sc_topk(scores):
    t = scores.shape[0]
    rows_per_w = t // _NW
    groups = rows_per_w // _L

    mesh = plsc.VectorSubcoreMesh(core_axis_name="c", subcore_axis_name="s")

    @functools.partial(
        pl.kernel,
        out_type=[
            jax.ShapeDtypeStruct((t, _L), jnp.float32),
            jax.ShapeDtypeStruct((t, _L), jnp.int32),
        ],
        mesh=mesh,
        scratch_types=[
            pltpu.VMEM((_L, _E), jnp.float32),
            pltpu.VMEM((_L, _L), jnp.float32),
            pltpu.VMEM((_L, _L), jnp.int32),
        ],
        compiler_params=pltpu.CompilerParams(needs_layout_passes=False),
    )
    def k(scores_hbm, wts_hbm, idx_hbm, sbuf, wbuf, ibuf):
        wid = lax.axis_index("s") * _NC + lax.axis_index("c")
        base = wid * rows_per_w
        lane = lax.iota(jnp.int32, _L)

        def body(g, carry):
            row0 = base + g * _L
            pltpu.sync_copy(scores_hbm.at[pl.ds(row0, _L), :], sbuf)
            for r in range(_L):
                mk = mv = None
                for j in range(_E // _L):
                    kj = sbuf[r, pl.ds(j * _L, _L)]
                    vj = lane + (j * _L)
                    sk, sv = plsc.sort_key_val(kj, vj, descending=True)
                    if mk is None:
                        mk, mv = sk, sv
                    else:
                        ck = jnp.where(lane < _TOPK, mk, lax.rev(sk, (0,)))
                        cv = jnp.where(lane < _TOPK, mv, lax.rev(sv, (0,)))
                        mk, mv = plsc.sort_key_val(ck, cv, descending=True)
                wbuf[r, :] = mk
                ibuf[r, :] = mv
            pltpu.sync_copy(wbuf, wts_hbm.at[pl.ds(row0, _L), :])
            pltpu.sync_copy(ibuf, idx_hbm.at[pl.ds(row0, _L), :])
            return carry

        lax.fori_loop(0, groups, body, 0)

    return k(scores)


def kernel(x, W):
    sl, bs, hs = x.shape
    t = sl * bs
    xf = x.reshape(t, hs)
    scores = _tc_scores(xf, W)
    wts16, idx16 = _sc_topk(scores)
    return scores, wts16[:, :_TOPK], idx16[:, :_TOPK]
